# baseline (speedup 1.0000x reference)
import functools

import jax
import jax.numpy as jnp
from jax import lax
from jax.experimental import pallas as pl
from jax.experimental.pallas import tpu as pltpu
from jax.experimental.pallas import tpu_sc as plsc

N = 10000
E = 320000
ATOM_DIM = 128
BOND_DIM = 16
H = 64
L = 3
NUM_GRAPHS = 64

NW = 32
EPT = E // NW
CH = 80
NCH = EPT // CH
NP = 10240
RPT = NP // 16

_F32 = jnp.float32


def _prologue_body(nf, Wn, bn, Wm0t, x_out, xm_out):
    x = jnp.dot(nf[...], Wn[...]) + bn[...]
    x_out[...] = x
    xm_out[...] = jnp.dot(x, Wm0t[...])


def _prologue(node_feats, Wn, bn, Wm0t):
    return pl.pallas_call(
        _prologue_body,
        out_shape=(
            jax.ShapeDtypeStruct((N, H), _F32),
            jax.ShapeDtypeStruct((N, H), _F32),
        ),
    )(node_feats, Wn, bn, Wm0t)


def _eam_body(efa, efb, We, be, Wmb, bm, out):
    dn = (((0,), (0,)), ((), ()))
    w = We[...]
    b = be[...]
    wm = Wmb[...]
    bb = bm[...]
    eaA = lax.dot_general(efa[...], w, dn,
                          preferred_element_type=_F32) + b
    eaB = lax.dot_general(efb[...], w, dn,
                          preferred_element_type=_F32) + b
    mA = jnp.dot(eaA, wm) + bb
    mB = jnp.dot(eaB, wm) + bb
    out[...] = jnp.concatenate([mA, mB], axis=1).astype(jnp.bfloat16)


def _eam_layer(eft, We, be, Wm_bot_l, bm_l):
    blk = 6400
    grid = (E // 2) // blk
    return pl.pallas_call(
        _eam_body,
        grid=(grid,),
        in_specs=[
            pl.BlockSpec((BOND_DIM, blk), lambda i: (0, i)),
            pl.BlockSpec((BOND_DIM, blk), lambda i, g=grid: (0, i + g)),
            pl.BlockSpec((BOND_DIM, H), lambda i: (0, 0)),
            pl.BlockSpec((1, H), lambda i: (0, 0)),
            pl.BlockSpec((H, H), lambda i: (0, 0)),
            pl.BlockSpec((1, H), lambda i: (0, 0)),
        ],
        out_specs=pl.BlockSpec((blk, 2 * H), lambda i: (i, 0)),
        out_shape=jax.ShapeDtypeStruct((E // 2, 2 * H), jnp.bfloat16),
    )(eft, eft, We, be, Wm_bot_l, bm_l)


def _update_body(with_xm, x, agg2, Wut, Wub, bu, g, b, Wmt, xo, xmo=None):
    agg = agg2[0] + agg2[1]
    xv = x[...]
    xn = jnp.dot(xv, Wut[...]) + jnp.dot(agg, Wub[...]) + bu[...]
    y = xv + xn
    mu = jnp.mean(y, axis=-1, keepdims=True)
    d = y - mu
    var = jnp.mean(d * d, axis=-1, keepdims=True)
    xh = d * lax.rsqrt(var + 1e-5) * g[...] + b[...]
    xo[...] = xh
    if with_xm:
        xmo[...] = jnp.dot(xh, Wmt[...])


def _update(x, agg2, Wut, Wub, bu, g, b, Wmt, with_xm):
    blk = 2000
    grid = N // blk
    out_shape = [jax.ShapeDtypeStruct((N, H), _F32)]
    out_specs = [pl.BlockSpec((blk, H), lambda i: (i, 0))]
    if with_xm:
        out_shape.append(jax.ShapeDtypeStruct((N, H), _F32))
        out_specs.append(pl.BlockSpec((blk, H), lambda i: (i, 0)))
    res = pl.pallas_call(
        functools.partial(_update_body, with_xm),
        grid=(grid,),
        in_specs=[
            pl.BlockSpec((blk, H), lambda i: (i, 0)),
            pl.BlockSpec((2, blk, H), lambda i: (0, i, 0)),
            pl.BlockSpec((H, H), lambda i: (0, 0)),
            pl.BlockSpec((H, H), lambda i: (0, 0)),
            pl.BlockSpec((1, H), lambda i: (0, 0)),
            pl.BlockSpec((1, H), lambda i: (0, 0)),
            pl.BlockSpec((1, H), lambda i: (0, 0)),
            pl.BlockSpec((H, H), lambda i: (0, 0)),
        ],
        out_specs=out_specs,
        out_shape=out_shape,
    )(x, agg2, Wut, Wub, bu, g, b, Wmt)
    return res if with_xm else (res[0], None)


def _pool_body(x, batch, Wp1, bp1, Wp2, bp2, out):
    gids = lax.broadcasted_iota(jnp.int32, (NUM_GRAPHS, 1), 0)
    A = (batch[...] == gids).astype(_F32)
    pooled = jnp.dot(A, x[...])
    counts = jnp.sum(A, axis=1, keepdims=True)
    pooled = pooled / jnp.maximum(counts, 1.0)
    h = jnp.maximum(jnp.dot(pooled, Wp1[...]) + bp1[...], 0.0)
    out[...] = jnp.dot(h, Wp2[...]) + bp2[...]


def _pool(x, batch2d, Wp1, bp1, Wp2, bp2):
    return pl.pallas_call(
        _pool_body,
        out_shape=jax.ShapeDtypeStruct((NUM_GRAPHS, H), _F32),
    )(x, batch2d, Wp1, bp1, Wp2, bp2)


NB = 5
LA = 2
ZR = 160


def _edge_pass_body(xm_hbm, eam_hbm, idx_hbm, out_hbm,
                    src_v, dst_v, *scr):
    bufs = scr[0:NB]
    gats = scr[NB:2 * NB]
    zbuf = scr[2 * NB]
    agg_sh = scr[2 * NB + 1]
    sems_e = scr[2 * NB + 2:2 * NB + 2 + NB]
    sems_g = scr[2 * NB + 2 + NB:2 * NB + 2 + 2 * NB]
    sems_s = scr[2 * NB + 2 + 2 * NB:2 * NB + 2 + 3 * NB]

    cc = lax.axis_index("c")
    ss = lax.axis_index("s")
    wid = cc * 16 + ss

    pltpu.sync_copy(idx_hbm.at[0, wid], src_v)
    pltpu.sync_copy(idx_hbm.at[1, wid], dst_v)

    ebase = wid * (EPT // 2)

    def start_loads(c, j):
        pltpu.async_copy(eam_hbm.at[pl.ds(ebase + c * (CH // 2), CH // 2)],
                         bufs[j], sems_e[j])
        pltpu.async_copy(xm_hbm.at[src_v.at[c]], gats[j], sems_g[j])

    start_loads(0, 0)
    start_loads(1, 1)

    def zrow(r, carry):
        for q in range(H // 16):
            zbuf[r, pl.ds(q * 16, 16)] = jnp.zeros((16,), _F32)
        return carry
    lax.fori_loop(0, ZR, zrow, 0)
    for q in range(RPT // ZR):
        pltpu.sync_copy(zbuf, agg_sh.at[pl.ds(ss * RPT + q * ZR, ZR)])
    plsc.subcore_barrier()

    def kbody(k, carry):
        for j in range(NB):
            c = NB * k + j
            pltpu.make_async_copy(
                eam_hbm.at[pl.ds(ebase + c * (CH // 2), CH // 2)], bufs[j],
                sems_e[j]).wait()
            pltpu.make_async_copy(xm_hbm.at[src_v.at[c]], gats[j], sems_g[j]).wait()

            buf, gm = bufs[j], gats[j]

            @plsc.parallel_loop(0, CH // 2, unroll=4)
            def _(r2):
                for p in range(2):
                    for q in range(H // 32):
                        v = buf[r2, pl.ds(p * H + q * 32, 32)]
                        a, b = plsc.unpack(
                            v, format=plsc.PackFormat.INTERLEAVED)
                        r = p * (CH // 2) + r2
                        s1 = pl.ds(q * 32, 16)
                        s2 = pl.ds(q * 32 + 16, 16)
                        gm[r, s1] = jnp.maximum(gm[r, s1] + a, 0.0)
                        gm[r, s2] = jnp.maximum(gm[r, s2] + b, 0.0)

            pltpu.async_copy(gm, agg_sh.at[dst_v.at[c]], sems_s[j], add=True)

            c2 = c + LA
            j2 = (j + LA) % NB

            @pl.when(c2 < NCH)
            def _():
                @pl.when(c2 >= NB)
                def _():
                    pltpu.make_async_copy(
                        bufs[j2], agg_sh.at[dst_v.at[c2]], sems_s[j2]).wait()
                start_loads(c2, j2)
        return carry
    lax.fori_loop(0, NCH // NB, kbody, 0)

    for j in range(NB):
        pltpu.make_async_copy(gats[j], agg_sh.at[dst_v.at[0]], sems_s[j]).wait()

    plsc.subcore_barrier()
    for q in range(RPT // ZR):
        pltpu.sync_copy(agg_sh.at[pl.ds(ss * RPT + q * ZR, ZR)],
                        out_hbm.at[cc, pl.ds(ss * RPT + q * ZR, ZR)])


def _edge_pass(xm, eam_l, idx4):
    mesh = plsc.VectorSubcoreMesh(core_axis_name="c", subcore_axis_name="s")
    kern = pl.kernel(
        _edge_pass_body,
        out_type=jax.ShapeDtypeStruct((2, NP, H), _F32),
        mesh=mesh,
        scratch_types=(
            [pltpu.VMEM((NCH, CH), jnp.int32)] * 2
            + [pltpu.VMEM((CH // 2, 2 * H), jnp.bfloat16)] * NB
            + [pltpu.VMEM((CH, H), _F32)] * NB
            + [pltpu.VMEM((ZR, H), _F32)]
            + [pltpu.VMEM_SHARED((NP, H), _F32)]
            + [pltpu.SemaphoreType.DMA] * (3 * NB)
        ),
        compiler_params=pltpu.CompilerParams(use_tc_tiling_on_sc=False, needs_layout_passes=False),
    )
    return kern(xm, eam_l, idx4)


def _perm():
    out = [0] * H
    for q in range(H // 32):
        for k in range(16):
            out[32 * q + k] = 32 * q + 2 * k
            out[32 * q + 16 + k] = 32 * q + 2 * k + 1
    return jnp.asarray(out, dtype=jnp.int32)


def kernel(node_feats, edge_feats, edge_index, batch,
           Wn, bn, We, be, Wm, bm, Wu, bu, ln_g, ln_b,
           Wp1, bp1, Wp2, bp2):
    perm = _perm()
    Wm_top = Wm[:, :H, :][:, :, perm]
    Wm_bot = Wm[:, H:, :]
    be2 = be.reshape(1, H)
    bn2 = bn.reshape(1, H)

    half = E // 2
    lo = edge_index[:, :half].reshape(2, NW, NCH, CH // 2)
    hi = edge_index[:, half:].reshape(2, NW, NCH, CH // 2)
    idx4 = jnp.concatenate([lo, hi], axis=3)
    batch2d = batch.reshape(1, N)
    eft = edge_feats.T

    x, xm = _prologue(node_feats, Wn, bn2, Wm_top[0])
    eams = [_eam_layer(eft, We, be2, Wm_bot[l], bm[l].reshape(1, H))
            for l in range(L)]

    for l in range(L):
        agg2 = _edge_pass(xm, eams[l], idx4)
        with_xm = l < L - 1
        Wmt = Wm_top[l + 1] if with_xm else Wm_top[0]
        x, xm = _update(
            x, agg2,
            Wu[l, :H, :], Wu[l, H:, :][perm, :], bu[l].reshape(1, H),
            ln_g[l].reshape(1, H), ln_b[l].reshape(1, H),
            Wmt, with_xm)

    return _pool(x, batch2d, Wp1.astype(_F32), bp1.reshape(1, H),
                 Wp2, bp2.reshape(1, H))


# speedup vs baseline: 1.5275x; 1.5275x over previous
import functools

import jax
import jax.numpy as jnp
from jax import lax
from jax.experimental import pallas as pl
from jax.experimental.pallas import tpu as pltpu
from jax.experimental.pallas import tpu_sc as plsc

N = 10000
E = 320000
ATOM_DIM = 128
BOND_DIM = 16
H = 64
L = 3
NUM_GRAPHS = 64

NW = 32
EPT = E // NW
CH = 80
NCH = EPT // CH
NP = 10240
RPT = NP // 16

_F32 = jnp.float32


def _prologue_body(nf, Wn, bn, Wm0t, x_out, xm_out):
    x = jnp.dot(nf[...], Wn[...]) + bn[...]
    x_out[...] = x
    xm_out[...] = jnp.dot(x, Wm0t[...])


def _prologue(node_feats, Wn, bn, Wm0t):
    return pl.pallas_call(
        _prologue_body,
        out_shape=(
            jax.ShapeDtypeStruct((N, H), _F32),
            jax.ShapeDtypeStruct((N, H), _F32),
        ),
    )(node_feats, Wn, bn, Wm0t)


def _rne16(x):
    xi = lax.bitcast_convert_type(x, jnp.uint32)
    return xi + jnp.uint32(0x7FFF) + (
        lax.shift_right_logical(xi, jnp.uint32(16)) & jnp.uint32(1))


def _eam_body(ef0, ef1, ef2, ef3, We, be, Wmb, bm, out):
    dn = (((0,), (0,)), ((), ()))
    w = We[...]
    b = be[...]
    wm = Wmb[...]
    bb = bm[...]
    efs = (ef0, ef1, ef2, ef3)
    packs = []
    for k in range(4):
        ea = lax.dot_general(efs[k][...], w, dn,
                             preferred_element_type=_F32) + b
        m = jnp.dot(ea, wm) + bb
        wlo = lax.shift_right_logical(_rne16(m[:, :H // 2]), jnp.uint32(16))
        whi = _rne16(m[:, H // 2:]) & jnp.uint32(0xFFFF0000)
        packs.append(wlo | whi)
    out[...] = jnp.concatenate(packs, axis=1)


def _eam_layer(eft, We, be, Wm_bot_l, bm_l):
    blk = 3200
    grid = (E // 4) // blk
    return pl.pallas_call(
        _eam_body,
        grid=(grid,),
        in_specs=[
            pl.BlockSpec((BOND_DIM, blk), lambda i, k=k, g=grid: (0, i + k * g))
            for k in range(4)
        ] + [
            pl.BlockSpec((BOND_DIM, H), lambda i: (0, 0)),
            pl.BlockSpec((1, H), lambda i: (0, 0)),
            pl.BlockSpec((H, H), lambda i: (0, 0)),
            pl.BlockSpec((1, H), lambda i: (0, 0)),
        ],
        out_specs=pl.BlockSpec((blk, 2 * H), lambda i: (i, 0)),
        out_shape=jax.ShapeDtypeStruct((E // 4, 2 * H), jnp.uint32),
    )(eft, eft, eft, eft, We, be, Wm_bot_l, bm_l)


def _update_body(with_xm, x, agg2, Wut, Wub, bu, g, b, Wmt, xo, xmo=None):
    agg = agg2[0] + agg2[1]
    xv = x[...]
    xn = jnp.dot(xv, Wut[...]) + jnp.dot(agg, Wub[...]) + bu[...]
    y = xv + xn
    mu = jnp.mean(y, axis=-1, keepdims=True)
    d = y - mu
    var = jnp.mean(d * d, axis=-1, keepdims=True)
    xh = d * lax.rsqrt(var + 1e-5) * g[...] + b[...]
    xo[...] = xh
    if with_xm:
        xmo[...] = jnp.dot(xh, Wmt[...])


def _update(x, agg2, Wut, Wub, bu, g, b, Wmt, with_xm):
    blk = 2000
    grid = N // blk
    out_shape = [jax.ShapeDtypeStruct((N, H), _F32)]
    out_specs = [pl.BlockSpec((blk, H), lambda i: (i, 0))]
    if with_xm:
        out_shape.append(jax.ShapeDtypeStruct((N, H), _F32))
        out_specs.append(pl.BlockSpec((blk, H), lambda i: (i, 0)))
    res = pl.pallas_call(
        functools.partial(_update_body, with_xm),
        grid=(grid,),
        in_specs=[
            pl.BlockSpec((blk, H), lambda i: (i, 0)),
            pl.BlockSpec((2, blk, H), lambda i: (0, i, 0)),
            pl.BlockSpec((H, H), lambda i: (0, 0)),
            pl.BlockSpec((H, H), lambda i: (0, 0)),
            pl.BlockSpec((1, H), lambda i: (0, 0)),
            pl.BlockSpec((1, H), lambda i: (0, 0)),
            pl.BlockSpec((1, H), lambda i: (0, 0)),
            pl.BlockSpec((H, H), lambda i: (0, 0)),
        ],
        out_specs=out_specs,
        out_shape=out_shape,
    )(x, agg2, Wut, Wub, bu, g, b, Wmt)
    return res if with_xm else (res[0], None)


def _pool_body(x, batch, Wp1, bp1, Wp2, bp2, out):
    gids = lax.broadcasted_iota(jnp.int32, (NUM_GRAPHS, 1), 0)
    A = (batch[...] == gids).astype(_F32)
    pooled = jnp.dot(A, x[...])
    counts = jnp.sum(A, axis=1, keepdims=True)
    pooled = pooled / jnp.maximum(counts, 1.0)
    h = jnp.maximum(jnp.dot(pooled, Wp1[...]) + bp1[...], 0.0)
    out[...] = jnp.dot(h, Wp2[...]) + bp2[...]


def _pool(x, batch2d, Wp1, bp1, Wp2, bp2):
    return pl.pallas_call(
        _pool_body,
        out_shape=jax.ShapeDtypeStruct((NUM_GRAPHS, H), _F32),
    )(x, batch2d, Wp1, bp1, Wp2, bp2)


NB = 5
LA = 2
ZR = 160


def _edge_pass_body(xm_hbm, eam_hbm, idx_hbm, out_hbm,
                    src_v, dst_v, *scr):
    bufs = scr[0:NB]
    gats = scr[NB:2 * NB]
    zbuf = scr[2 * NB]
    agg_sh = scr[2 * NB + 1]
    sems_e = scr[2 * NB + 2:2 * NB + 2 + NB]
    sems_g = scr[2 * NB + 2 + NB:2 * NB + 2 + 2 * NB]
    sems_s = scr[2 * NB + 2 + 2 * NB:2 * NB + 2 + 3 * NB]

    cc = lax.axis_index("c")
    ss = lax.axis_index("s")
    wid = cc * 16 + ss

    pltpu.sync_copy(idx_hbm.at[0, wid], src_v)
    pltpu.sync_copy(idx_hbm.at[1, wid], dst_v)

    ebase = wid * (EPT // 4)

    def start_loads(c, j):
        pltpu.async_copy(eam_hbm.at[pl.ds(ebase + c * (CH // 4), CH // 4)],
                         bufs[j], sems_e[j])
        pltpu.async_copy(xm_hbm.at[src_v.at[c]], gats[j], sems_g[j])

    start_loads(0, 0)
    start_loads(1, 1)

    def zrow(r, carry):
        for q in range(H // 16):
            zbuf[r, pl.ds(q * 16, 16)] = jnp.zeros((16,), _F32)
        return carry
    lax.fori_loop(0, ZR, zrow, 0)
    for q in range(RPT // ZR):
        pltpu.sync_copy(zbuf, agg_sh.at[pl.ds(ss * RPT + q * ZR, ZR)])
    plsc.subcore_barrier()

    def kbody(k, carry):
        for j in range(NB):
            c = NB * k + j
            pltpu.make_async_copy(
                eam_hbm.at[pl.ds(ebase + c * (CH // 4), CH // 4)], bufs[j],
                sems_e[j]).wait()
            pltpu.make_async_copy(xm_hbm.at[src_v.at[c]], gats[j], sems_g[j]).wait()

            buf, gm = bufs[j], gats[j]

            @plsc.parallel_loop(0, CH // 4, unroll=4)
            def _(r4):
                for k in range(4):
                    for q in range(H // 32):
                        v = buf[r4, pl.ds(k * 32 + q * 16, 16)]
                        a, b = plsc.unpack(
                            plsc.bitcast(v, jnp.bfloat16),
                            format=plsc.PackFormat.INTERLEAVED)
                        r = k * (CH // 4) + r4
                        s1 = pl.ds(q * 32, 16)
                        s2 = pl.ds(q * 32 + 16, 16)
                        gm[r, s1] = jnp.maximum(gm[r, s1] + a, 0.0)
                        gm[r, s2] = jnp.maximum(gm[r, s2] + b, 0.0)

            pltpu.async_copy(gm, agg_sh.at[dst_v.at[c]], sems_s[j], add=True)

            c2 = c + LA
            j2 = (j + LA) % NB

            @pl.when(c2 < NCH)
            def _():
                @pl.when(c2 >= NB)
                def _():
                    pltpu.make_async_copy(
                        bufs[j2], agg_sh.at[dst_v.at[c2]], sems_s[j2]).wait()
                start_loads(c2, j2)
        return carry
    lax.fori_loop(0, NCH // NB, kbody, 0)

    for j in range(NB):
        pltpu.make_async_copy(gats[j], agg_sh.at[dst_v.at[0]], sems_s[j]).wait()

    plsc.subcore_barrier()
    for q in range(RPT // ZR):
        pltpu.sync_copy(agg_sh.at[pl.ds(ss * RPT + q * ZR, ZR)],
                        out_hbm.at[cc, pl.ds(ss * RPT + q * ZR, ZR)])


def _edge_pass(xm, eam_l, idx4):
    mesh = plsc.VectorSubcoreMesh(core_axis_name="c", subcore_axis_name="s")
    kern = pl.kernel(
        _edge_pass_body,
        out_type=jax.ShapeDtypeStruct((2, NP, H), _F32),
        mesh=mesh,
        scratch_types=(
            [pltpu.VMEM((NCH, CH), jnp.int32)] * 2
            + [pltpu.VMEM((CH // 4, 2 * H), jnp.uint32)] * NB
            + [pltpu.VMEM((CH, H), _F32)] * NB
            + [pltpu.VMEM((ZR, H), _F32)]
            + [pltpu.VMEM_SHARED((NP, H), _F32)]
            + [pltpu.SemaphoreType.DMA] * (3 * NB)
        ),
        compiler_params=pltpu.CompilerParams(use_tc_tiling_on_sc=False, needs_layout_passes=False),
    )
    return kern(xm, eam_l, idx4)


def _psi():
    return jnp.asarray(
        list(range(0, 16)) + list(range(32, 48))
        + list(range(16, 32)) + list(range(48, 64)), dtype=jnp.int32)


def kernel(node_feats, edge_feats, edge_index, batch,
           Wn, bn, We, be, Wm, bm, Wu, bu, ln_g, ln_b,
           Wp1, bp1, Wp2, bp2):
    psi = _psi()
    Wm_top = Wm[:, :H, :]
    Wm_bot = Wm[:, H:, :][:, :, psi]
    bm_p = bm[:, psi]
    be2 = be.reshape(1, H)
    bn2 = bn.reshape(1, H)

    quarter = E // 4
    slabs = [edge_index[:, k * quarter:(k + 1) * quarter]
             .reshape(2, NW, NCH, CH // 4) for k in range(4)]
    idx4 = jnp.concatenate(slabs, axis=3)
    batch2d = batch.reshape(1, N)
    eft = edge_feats.T

    x, xm = _prologue(node_feats, Wn, bn2, Wm_top[0])
    eams = [_eam_layer(eft, We, be2, Wm_bot[l], bm_p[l].reshape(1, H))
            for l in range(L)]

    for l in range(L):
        agg2 = _edge_pass(xm, eams[l], idx4)
        with_xm = l < L - 1
        Wmt = Wm_top[l + 1] if with_xm else Wm_top[0]
        x, xm = _update(
            x, agg2,
            Wu[l, :H, :], Wu[l, H:, :], bu[l].reshape(1, H),
            ln_g[l].reshape(1, H), ln_b[l].reshape(1, H),
            Wmt, with_xm)

    return _pool(x, batch2d, Wp1.astype(_F32), bp1.reshape(1, H),
                 Wp2, bp2.reshape(1, H))


# speedup vs baseline: 1.5364x; 1.0058x over previous
import functools

import jax
import jax.numpy as jnp
from jax import lax
from jax.experimental import pallas as pl
from jax.experimental.pallas import tpu as pltpu
from jax.experimental.pallas import tpu_sc as plsc

N = 10000
E = 320000
ATOM_DIM = 128
BOND_DIM = 16
H = 64
L = 3
NUM_GRAPHS = 64

NW = 32
EPT = E // NW
CH = 80
NCH = EPT // CH
NP = 10240
RPT = NP // 16

_F32 = jnp.float32


def _prologue_body(nf, Wn, bn, Wm0t, x_out, xm_out):
    x = jnp.dot(nf[...], Wn[...]) + bn[...]
    x_out[...] = x
    xm_out[...] = jnp.dot(x, Wm0t[...])


def _prologue(node_feats, Wn, bn, Wm0t):
    return pl.pallas_call(
        _prologue_body,
        out_shape=(
            jax.ShapeDtypeStruct((N, H), _F32),
            jax.ShapeDtypeStruct((N, H), _F32),
        ),
    )(node_feats, Wn, bn, Wm0t)


def _rn16(x):
    xi = lax.bitcast_convert_type(x, jnp.uint32)
    return xi + jnp.uint32(0x8000)


def _eam_body(ef0, ef1, ef2, ef3, Wf, bf, out):
    dn = (((0,), (0,)), ((), ()))
    w = Wf[...]
    b = bf[...]
    efs = (ef0, ef1, ef2, ef3)
    packs = []
    for k in range(4):
        m = lax.dot_general(efs[k][...], w, dn,
                            preferred_element_type=_F32) + b
        wlo = lax.shift_right_logical(_rn16(m[:, :H // 2]), jnp.uint32(16))
        whi = _rn16(m[:, H // 2:]) & jnp.uint32(0xFFFF0000)
        packs.append(wlo | whi)
    out[...] = jnp.concatenate(packs, axis=1)


def _eam_layer(eft, Wf_l, bf_l):
    blk = 3200
    grid = (E // 4) // blk
    return pl.pallas_call(
        _eam_body,
        grid=(grid,),
        in_specs=[
            pl.BlockSpec((BOND_DIM, blk), lambda i, k=k, g=grid: (0, i + k * g))
            for k in range(4)
        ] + [
            pl.BlockSpec((BOND_DIM, H), lambda i: (0, 0)),
            pl.BlockSpec((1, H), lambda i: (0, 0)),
        ],
        out_specs=pl.BlockSpec((blk, 2 * H), lambda i: (i, 0)),
        out_shape=jax.ShapeDtypeStruct((E // 4, 2 * H), jnp.uint32),
    )(eft, eft, eft, eft, Wf_l, bf_l)


def _update_body(with_xm, x, agg2, Wut, Wub, bu, g, b, Wmt, xo, xmo=None):
    agg = agg2[0] + agg2[1]
    xv = x[...]
    xn = jnp.dot(xv, Wut[...]) + jnp.dot(agg, Wub[...]) + bu[...]
    y = xv + xn
    mu = jnp.mean(y, axis=-1, keepdims=True)
    d = y - mu
    var = jnp.mean(d * d, axis=-1, keepdims=True)
    xh = d * lax.rsqrt(var + 1e-5) * g[...] + b[...]
    xo[...] = xh
    if with_xm:
        xmo[...] = jnp.dot(xh, Wmt[...])


def _update(x, agg2, Wut, Wub, bu, g, b, Wmt, with_xm):
    blk = 2000
    grid = N // blk
    out_shape = [jax.ShapeDtypeStruct((N, H), _F32)]
    out_specs = [pl.BlockSpec((blk, H), lambda i: (i, 0))]
    if with_xm:
        out_shape.append(jax.ShapeDtypeStruct((N, H), _F32))
        out_specs.append(pl.BlockSpec((blk, H), lambda i: (i, 0)))
    res = pl.pallas_call(
        functools.partial(_update_body, with_xm),
        grid=(grid,),
        in_specs=[
            pl.BlockSpec((blk, H), lambda i: (i, 0)),
            pl.BlockSpec((2, blk, H), lambda i: (0, i, 0)),
            pl.BlockSpec((H, H), lambda i: (0, 0)),
            pl.BlockSpec((H, H), lambda i: (0, 0)),
            pl.BlockSpec((1, H), lambda i: (0, 0)),
            pl.BlockSpec((1, H), lambda i: (0, 0)),
            pl.BlockSpec((1, H), lambda i: (0, 0)),
            pl.BlockSpec((H, H), lambda i: (0, 0)),
        ],
        out_specs=out_specs,
        out_shape=out_shape,
    )(x, agg2, Wut, Wub, bu, g, b, Wmt)
    return res if with_xm else (res[0], None)


def _pool_body(x, batch, Wp1, bp1, Wp2, bp2, out):
    gids = lax.broadcasted_iota(jnp.int32, (NUM_GRAPHS, 1), 0)
    A = (batch[...] == gids).astype(_F32)
    pooled = jnp.dot(A, x[...])
    counts = jnp.sum(A, axis=1, keepdims=True)
    pooled = pooled / jnp.maximum(counts, 1.0)
    h = jnp.maximum(jnp.dot(pooled, Wp1[...]) + bp1[...], 0.0)
    out[...] = jnp.dot(h, Wp2[...]) + bp2[...]


def _pool(x, batch2d, Wp1, bp1, Wp2, bp2):
    return pl.pallas_call(
        _pool_body,
        out_shape=jax.ShapeDtypeStruct((NUM_GRAPHS, H), _F32),
    )(x, batch2d, Wp1, bp1, Wp2, bp2)


NB = 5
LA = 2
ZR = 160


def _edge_pass_body(xm_hbm, eam_hbm, idx_hbm, out_hbm,
                    src_v, dst_v, *scr):
    bufs = scr[0:NB]
    gats = scr[NB:2 * NB]
    zbuf = scr[2 * NB]
    agg_sh = scr[2 * NB + 1]
    sems_e = scr[2 * NB + 2:2 * NB + 2 + NB]
    sems_g = scr[2 * NB + 2 + NB:2 * NB + 2 + 2 * NB]
    sems_s = scr[2 * NB + 2 + 2 * NB:2 * NB + 2 + 3 * NB]

    cc = lax.axis_index("c")
    ss = lax.axis_index("s")
    wid = cc * 16 + ss

    pltpu.sync_copy(idx_hbm.at[0, wid], src_v)
    pltpu.sync_copy(idx_hbm.at[1, wid], dst_v)

    ebase = wid * (EPT // 4)

    def start_loads(c, j):
        pltpu.async_copy(eam_hbm.at[pl.ds(ebase + c * (CH // 4), CH // 4)],
                         bufs[j], sems_e[j])
        pltpu.async_copy(xm_hbm.at[src_v.at[c]], gats[j], sems_g[j])

    start_loads(0, 0)
    start_loads(1, 1)

    def zrow(r, carry):
        for q in range(H // 16):
            zbuf[r, pl.ds(q * 16, 16)] = jnp.zeros((16,), _F32)
        return carry
    lax.fori_loop(0, ZR, zrow, 0)
    for q in range(RPT // ZR):
        pltpu.sync_copy(zbuf, agg_sh.at[pl.ds(ss * RPT + q * ZR, ZR)])
    plsc.subcore_barrier()

    def kbody(k, carry):
        for j in range(NB):
            c = NB * k + j
            pltpu.make_async_copy(
                eam_hbm.at[pl.ds(ebase + c * (CH // 4), CH // 4)], bufs[j],
                sems_e[j]).wait()
            pltpu.make_async_copy(xm_hbm.at[src_v.at[c]], gats[j], sems_g[j]).wait()

            buf, gm = bufs[j], gats[j]

            @plsc.parallel_loop(0, CH // 4, unroll=4)
            def _(r4):
                for k in range(4):
                    for q in range(H // 32):
                        v = buf[r4, pl.ds(k * 32 + q * 16, 16)]
                        a, b = plsc.unpack(
                            plsc.bitcast(v, jnp.bfloat16),
                            format=plsc.PackFormat.INTERLEAVED)
                        r = k * (CH // 4) + r4
                        s1 = pl.ds(q * 32, 16)
                        s2 = pl.ds(q * 32 + 16, 16)
                        gm[r, s1] = jnp.maximum(gm[r, s1] + a, 0.0)
                        gm[r, s2] = jnp.maximum(gm[r, s2] + b, 0.0)

            pltpu.async_copy(gm, agg_sh.at[dst_v.at[c]], sems_s[j], add=True)

            c2 = c + LA
            j2 = (j + LA) % NB

            @pl.when(c2 < NCH)
            def _():
                @pl.when(c2 >= NB)
                def _():
                    pltpu.make_async_copy(
                        bufs[j2], agg_sh.at[dst_v.at[c2]], sems_s[j2]).wait()
                start_loads(c2, j2)
        return carry
    lax.fori_loop(0, NCH // NB, kbody, 0)

    for j in range(NB):
        pltpu.make_async_copy(gats[j], agg_sh.at[dst_v.at[0]], sems_s[j]).wait()

    plsc.subcore_barrier()
    for q in range(RPT // ZR):
        pltpu.sync_copy(agg_sh.at[pl.ds(ss * RPT + q * ZR, ZR)],
                        out_hbm.at[cc, pl.ds(ss * RPT + q * ZR, ZR)])


def _edge_pass(xm, eam_l, idx4):
    mesh = plsc.VectorSubcoreMesh(core_axis_name="c", subcore_axis_name="s")
    kern = pl.kernel(
        _edge_pass_body,
        out_type=jax.ShapeDtypeStruct((2, NP, H), _F32),
        mesh=mesh,
        scratch_types=(
            [pltpu.VMEM((NCH, CH), jnp.int32)] * 2
            + [pltpu.VMEM((CH // 4, 2 * H), jnp.uint32)] * NB
            + [pltpu.VMEM((CH, H), _F32)] * NB
            + [pltpu.VMEM((ZR, H), _F32)]
            + [pltpu.VMEM_SHARED((NP, H), _F32)]
            + [pltpu.SemaphoreType.DMA] * (3 * NB)
        ),
        compiler_params=pltpu.CompilerParams(use_tc_tiling_on_sc=False, needs_layout_passes=False),
    )
    return kern(xm, eam_l, idx4)


def _psi():
    return jnp.asarray(
        list(range(0, 16)) + list(range(32, 48))
        + list(range(16, 32)) + list(range(48, 64)), dtype=jnp.int32)


def kernel(node_feats, edge_feats, edge_index, batch,
           Wn, bn, We, be, Wm, bm, Wu, bu, ln_g, ln_b,
           Wp1, bp1, Wp2, bp2):
    psi = _psi()
    Wm_top = Wm[:, :H, :]
    Wm_bot = Wm[:, H:, :][:, :, psi]
    bm_p = bm[:, psi]
    Wf = jnp.einsum("kh,lhj->lkj", We, Wm_bot)
    bf = jnp.einsum("h,lhj->lj", be, Wm_bot) + bm_p
    be2 = be.reshape(1, H)
    bn2 = bn.reshape(1, H)

    quarter = E // 4
    slabs = [edge_index[:, k * quarter:(k + 1) * quarter]
             .reshape(2, NW, NCH, CH // 4) for k in range(4)]
    idx4 = jnp.concatenate(slabs, axis=3)
    batch2d = batch.reshape(1, N)
    eft = edge_feats.T

    x, xm = _prologue(node_feats, Wn, bn2, Wm_top[0])
    eams = [_eam_layer(eft, Wf[l], bf[l].reshape(1, H)) for l in range(L)]

    for l in range(L):
        agg2 = _edge_pass(xm, eams[l], idx4)
        with_xm = l < L - 1
        Wmt = Wm_top[l + 1] if with_xm else Wm_top[0]
        x, xm = _update(
            x, agg2,
            Wu[l, :H, :], Wu[l, H:, :], bu[l].reshape(1, H),
            ln_g[l].reshape(1, H), ln_b[l].reshape(1, H),
            Wmt, with_xm)

    return _pool(x, batch2d, Wp1.astype(_F32), bp1.reshape(1, H),
                 Wp2, bp2.reshape(1, H))


# speedup vs baseline: 1.8225x; 1.1863x over previous
import functools

import jax
import jax.numpy as jnp
from jax import lax
from jax.experimental import pallas as pl
from jax.experimental.pallas import tpu as pltpu
from jax.experimental.pallas import tpu_sc as plsc

N = 10000
E = 320000
ATOM_DIM = 128
BOND_DIM = 16
H = 64
L = 3
NUM_GRAPHS = 64

NW = 32
EPT = E // NW
CH = 80
NCH = EPT // CH
NP = 10240
RPT = NP // 16

_F32 = jnp.float32


def _prologue_body(nf, Wn, bn, Wm0t, x_out, xm_out):
    x = jnp.dot(nf[...], Wn[...]) + bn[...]
    x_out[...] = x
    xm_out[...] = jnp.dot(x, Wm0t[...])


def _prologue(node_feats, Wn, bn, Wm0t):
    return pl.pallas_call(
        _prologue_body,
        out_shape=(
            jax.ShapeDtypeStruct((N, H), _F32),
            jax.ShapeDtypeStruct((N, H), _F32),
        ),
    )(node_feats, Wn, bn, Wm0t)


def _rn16(x):
    xi = lax.bitcast_convert_type(x, jnp.uint32)
    return xi + jnp.uint32(0x8000)


def _eam_body(ef0, ef1, ef2, ef3, Wf4, bf4, out):
    dn = (((0,), (0,)), ((), ()))
    lhs = jnp.concatenate([ef0[...], ef1[...], ef2[...], ef3[...]], axis=0)
    m4 = lax.dot_general(lhs, Wf4[...], dn,
                         preferred_element_type=_F32) + bf4[...]
    wlo = lax.shift_right_logical(_rn16(m4[:, :2 * H]), jnp.uint32(16))
    whi = _rn16(m4[:, 2 * H:]) & jnp.uint32(0xFFFF0000)
    out[...] = wlo | whi


def _eam_layer(eft, Wf_l, bf_l):
    blk = 3200
    grid = (E // 4) // blk
    return pl.pallas_call(
        _eam_body,
        grid=(grid,),
        in_specs=[
            pl.BlockSpec((BOND_DIM, blk), lambda i, k=k, g=grid: (0, i + k * g))
            for k in range(4)
        ] + [
            pl.BlockSpec((4 * BOND_DIM, 4 * H), lambda i: (0, 0)),
            pl.BlockSpec((1, 4 * H), lambda i: (0, 0)),
        ],
        out_specs=pl.BlockSpec((blk, 2 * H), lambda i: (i, 0)),
        out_shape=jax.ShapeDtypeStruct((E // 4, 2 * H), jnp.uint32),
    )(eft, eft, eft, eft, Wf_l, bf_l)


def _update_body(with_xm, x, agg2, Wut, Wub, bu, g, b, Wmt, xo, xmo=None):
    agg = agg2[0] + agg2[1]
    xv = x[...]
    xn = jnp.dot(xv, Wut[...]) + jnp.dot(agg, Wub[...]) + bu[...]
    y = xv + xn
    mu = jnp.mean(y, axis=-1, keepdims=True)
    d = y - mu
    var = jnp.mean(d * d, axis=-1, keepdims=True)
    xh = d * lax.rsqrt(var + 1e-5) * g[...] + b[...]
    xo[...] = xh
    if with_xm:
        xmo[...] = jnp.dot(xh, Wmt[...])


def _update(x, agg2, Wut, Wub, bu, g, b, Wmt, with_xm):
    blk = 2000
    grid = N // blk
    out_shape = [jax.ShapeDtypeStruct((N, H), _F32)]
    out_specs = [pl.BlockSpec((blk, H), lambda i: (i, 0))]
    if with_xm:
        out_shape.append(jax.ShapeDtypeStruct((N, H), _F32))
        out_specs.append(pl.BlockSpec((blk, H), lambda i: (i, 0)))
    res = pl.pallas_call(
        functools.partial(_update_body, with_xm),
        grid=(grid,),
        in_specs=[
            pl.BlockSpec((blk, H), lambda i: (i, 0)),
            pl.BlockSpec((2, blk, H), lambda i: (0, i, 0)),
            pl.BlockSpec((H, H), lambda i: (0, 0)),
            pl.BlockSpec((H, H), lambda i: (0, 0)),
            pl.BlockSpec((1, H), lambda i: (0, 0)),
            pl.BlockSpec((1, H), lambda i: (0, 0)),
            pl.BlockSpec((1, H), lambda i: (0, 0)),
            pl.BlockSpec((H, H), lambda i: (0, 0)),
        ],
        out_specs=out_specs,
        out_shape=out_shape,
    )(x, agg2, Wut, Wub, bu, g, b, Wmt)
    return res if with_xm else (res[0], None)


def _pool_body(x, batch, Wp1, bp1, Wp2, bp2, out):
    gids = lax.broadcasted_iota(jnp.int32, (NUM_GRAPHS, 1), 0)
    A = (batch[...] == gids).astype(_F32)
    pooled = jnp.dot(A, x[...])
    counts = jnp.sum(A, axis=1, keepdims=True)
    pooled = pooled / jnp.maximum(counts, 1.0)
    h = jnp.maximum(jnp.dot(pooled, Wp1[...]) + bp1[...], 0.0)
    out[...] = jnp.dot(h, Wp2[...]) + bp2[...]


def _pool(x, batch2d, Wp1, bp1, Wp2, bp2):
    return pl.pallas_call(
        _pool_body,
        out_shape=jax.ShapeDtypeStruct((NUM_GRAPHS, H), _F32),
    )(x, batch2d, Wp1, bp1, Wp2, bp2)


NB = 5
LA = 2
ZR = 160


def _edge_pass_body(xm_hbm, eam_hbm, idx_hbm, out_hbm,
                    src_v, dst_v, *scr):
    bufs = scr[0:NB]
    gats = scr[NB:2 * NB]
    zbuf = scr[2 * NB]
    agg_sh = scr[2 * NB + 1]
    sems_e = scr[2 * NB + 2:2 * NB + 2 + NB]
    sems_g = scr[2 * NB + 2 + NB:2 * NB + 2 + 2 * NB]
    sems_s = scr[2 * NB + 2 + 2 * NB:2 * NB + 2 + 3 * NB]

    cc = lax.axis_index("c")
    ss = lax.axis_index("s")
    wid = cc * 16 + ss

    pltpu.sync_copy(idx_hbm.at[0, wid], src_v)
    pltpu.sync_copy(idx_hbm.at[1, wid], dst_v)

    ebase = wid * (EPT // 4)

    def start_loads(c, j):
        pltpu.async_copy(eam_hbm.at[pl.ds(ebase + c * (CH // 4), CH // 4)],
                         bufs[j], sems_e[j])
        pltpu.async_copy(xm_hbm.at[src_v.at[c]], gats[j], sems_g[j])

    start_loads(0, 0)
    start_loads(1, 1)

    def zrow(r, carry):
        for q in range(H // 16):
            zbuf[r, pl.ds(q * 16, 16)] = jnp.zeros((16,), _F32)
        return carry
    lax.fori_loop(0, ZR, zrow, 0)
    for q in range(RPT // ZR):
        pltpu.sync_copy(zbuf, agg_sh.at[pl.ds(ss * RPT + q * ZR, ZR)])
    plsc.subcore_barrier()

    def kbody(k, carry):
        for j in range(NB):
            c = NB * k + j
            pltpu.make_async_copy(
                eam_hbm.at[pl.ds(ebase + c * (CH // 4), CH // 4)], bufs[j],
                sems_e[j]).wait()
            pltpu.make_async_copy(xm_hbm.at[src_v.at[c]], gats[j], sems_g[j]).wait()

            buf, gm = bufs[j], gats[j]

            @plsc.parallel_loop(0, CH // 4, unroll=4)
            def _(r4):
                for k in range(4):
                    for q in range(H // 32):
                        v = buf[r4, pl.ds(k * 32 + q * 16, 16)]
                        a, b = plsc.unpack(
                            plsc.bitcast(v, jnp.bfloat16),
                            format=plsc.PackFormat.INTERLEAVED)
                        r = k * (CH // 4) + r4
                        s1 = pl.ds(q * 32, 16)
                        s2 = pl.ds(q * 32 + 16, 16)
                        gm[r, s1] = jnp.maximum(gm[r, s1] + a, 0.0)
                        gm[r, s2] = jnp.maximum(gm[r, s2] + b, 0.0)

            pltpu.async_copy(gm, agg_sh.at[dst_v.at[c]], sems_s[j], add=True)

            c2 = c + LA
            j2 = (j + LA) % NB

            @pl.when(c2 < NCH)
            def _():
                @pl.when(c2 >= NB)
                def _():
                    pltpu.make_async_copy(
                        bufs[j2], agg_sh.at[dst_v.at[c2]], sems_s[j2]).wait()
                start_loads(c2, j2)
        return carry
    lax.fori_loop(0, NCH // NB, kbody, 0)

    for j in range(NB):
        pltpu.make_async_copy(gats[j], agg_sh.at[dst_v.at[0]], sems_s[j]).wait()

    plsc.subcore_barrier()
    for q in range(RPT // ZR):
        pltpu.sync_copy(agg_sh.at[pl.ds(ss * RPT + q * ZR, ZR)],
                        out_hbm.at[cc, pl.ds(ss * RPT + q * ZR, ZR)])


def _edge_pass(xm, eam_l, idx4):
    mesh = plsc.VectorSubcoreMesh(core_axis_name="c", subcore_axis_name="s")
    kern = pl.kernel(
        _edge_pass_body,
        out_type=jax.ShapeDtypeStruct((2, NP, H), _F32),
        mesh=mesh,
        scratch_types=(
            [pltpu.VMEM((NCH, CH), jnp.int32)] * 2
            + [pltpu.VMEM((CH // 4, 2 * H), jnp.uint32)] * NB
            + [pltpu.VMEM((CH, H), _F32)] * NB
            + [pltpu.VMEM((ZR, H), _F32)]
            + [pltpu.VMEM_SHARED((NP, H), _F32)]
            + [pltpu.SemaphoreType.DMA] * (3 * NB)
        ),
        compiler_params=pltpu.CompilerParams(use_tc_tiling_on_sc=False, needs_layout_passes=False),
    )
    return kern(xm, eam_l, idx4)


def _psi():
    return jnp.asarray(
        list(range(0, 16)) + list(range(32, 48))
        + list(range(16, 32)) + list(range(48, 64)), dtype=jnp.int32)


def kernel(node_feats, edge_feats, edge_index, batch,
           Wn, bn, We, be, Wm, bm, Wu, bu, ln_g, ln_b,
           Wp1, bp1, Wp2, bp2):
    psi = _psi()
    Wm_top = Wm[:, :H, :]
    Wm_bot = Wm[:, H:, :][:, :, psi]
    bm_p = bm[:, psi]
    import jax.scipy.linalg as jsl
    Wfold = jnp.einsum("kh,lhj->lkj", We, Wm_bot)
    bfold = jnp.einsum("h,lhj->lj", be, Wm_bot) + bm_p
    Wf4, bf4 = [], []
    for l in range(L):
        lo = Wfold[l][:, :H // 2]
        hi = Wfold[l][:, H // 2:]
        Wf4.append(jnp.concatenate(
            [jsl.block_diag(lo, lo, lo, lo), jsl.block_diag(hi, hi, hi, hi)],
            axis=1))
        bf4.append(jnp.concatenate(
            [jnp.tile(bfold[l][:H // 2], 4), jnp.tile(bfold[l][H // 2:], 4)]))
    be2 = be.reshape(1, H)
    bn2 = bn.reshape(1, H)

    quarter = E // 4
    slabs = [edge_index[:, k * quarter:(k + 1) * quarter]
             .reshape(2, NW, NCH, CH // 4) for k in range(4)]
    idx4 = jnp.concatenate(slabs, axis=3)
    batch2d = batch.reshape(1, N)
    eft = edge_feats.T

    x, xm = _prologue(node_feats, Wn, bn2, Wm_top[0])
    eams = [_eam_layer(eft, Wf4[l], bf4[l].reshape(1, 4 * H))
            for l in range(L)]

    for l in range(L):
        agg2 = _edge_pass(xm, eams[l], idx4)
        with_xm = l < L - 1
        Wmt = Wm_top[l + 1] if with_xm else Wm_top[0]
        x, xm = _update(
            x, agg2,
            Wu[l, :H, :], Wu[l, H:, :], bu[l].reshape(1, H),
            ln_g[l].reshape(1, H), ln_b[l].reshape(1, H),
            Wmt, with_xm)

    return _pool(x, batch2d, Wp1.astype(_F32), bp1.reshape(1, H),
                 Wp2, bp2.reshape(1, H))


# speedup vs baseline: 1.8259x; 1.0019x over previous
import functools

import jax
import jax.numpy as jnp
from jax import lax
from jax.experimental import pallas as pl
from jax.experimental.pallas import tpu as pltpu
from jax.experimental.pallas import tpu_sc as plsc

N = 10000
E = 320000
ATOM_DIM = 128
BOND_DIM = 16
H = 64
L = 3
NUM_GRAPHS = 64

NW = 32
EPT = E // NW
CH = 80
NCH = EPT // CH
NP = 10240
RPT = NP // 16

_F32 = jnp.float32


def _prologue_body(nf, Wn, bn, Wm0t, x_out, xm_out):
    x = jnp.dot(nf[...], Wn[...]) + bn[...]
    x_out[...] = x
    xm_out[...] = _pack32(jnp.dot(x, Wm0t[...]))


def _prologue(node_feats, Wn, bn, Wm0t):
    return pl.pallas_call(
        _prologue_body,
        out_shape=(
            jax.ShapeDtypeStruct((N, H), _F32),
            jax.ShapeDtypeStruct((N, H // 2), jnp.uint32),
        ),
    )(node_feats, Wn, bn, Wm0t)


def _rn16(x):
    xi = lax.bitcast_convert_type(x, jnp.uint32)
    return xi + jnp.uint32(0x8000)


def _pack32(m):
    wlo = lax.shift_right_logical(_rn16(m[:, :H // 2]), jnp.uint32(16))
    return wlo | (_rn16(m[:, H // 2:]) & jnp.uint32(0xFFFF0000))


def _eam_body(ef0, ef1, ef2, ef3, Wf4, bf4, out):
    dn = (((0,), (0,)), ((), ()))
    lhs = jnp.concatenate([ef0[...], ef1[...], ef2[...], ef3[...]], axis=0)
    m4 = lax.dot_general(lhs, Wf4[...], dn,
                         preferred_element_type=_F32) + bf4[...]
    wlo = lax.shift_right_logical(_rn16(m4[:, :2 * H]), jnp.uint32(16))
    whi = _rn16(m4[:, 2 * H:]) & jnp.uint32(0xFFFF0000)
    out[...] = wlo | whi


def _eam_layer(eft, Wf_l, bf_l):
    blk = 3200
    grid = (E // 4) // blk
    return pl.pallas_call(
        _eam_body,
        grid=(grid,),
        in_specs=[
            pl.BlockSpec((BOND_DIM, blk), lambda i, k=k, g=grid: (0, i + k * g))
            for k in range(4)
        ] + [
            pl.BlockSpec((4 * BOND_DIM, 4 * H), lambda i: (0, 0)),
            pl.BlockSpec((1, 4 * H), lambda i: (0, 0)),
        ],
        out_specs=pl.BlockSpec((blk, 2 * H), lambda i: (i, 0)),
        out_shape=jax.ShapeDtypeStruct((E // 4, 2 * H), jnp.uint32),
    )(eft, eft, eft, eft, Wf_l, bf_l)


def _update_body(with_xm, x, agg2, Wut, Wub, bu, g, b, Wmt, xo, xmo=None):
    agg = agg2[0] + agg2[1]
    xv = x[...]
    xn = jnp.dot(xv, Wut[...]) + jnp.dot(agg, Wub[...]) + bu[...]
    y = xv + xn
    mu = jnp.mean(y, axis=-1, keepdims=True)
    d = y - mu
    var = jnp.mean(d * d, axis=-1, keepdims=True)
    xh = d * lax.rsqrt(var + 1e-5) * g[...] + b[...]
    xo[...] = xh
    if with_xm:
        xmo[...] = _pack32(jnp.dot(xh, Wmt[...]))


def _update(x, agg2, Wut, Wub, bu, g, b, Wmt, with_xm):
    blk = 2000
    grid = N // blk
    out_shape = [jax.ShapeDtypeStruct((N, H), _F32)]
    out_specs = [pl.BlockSpec((blk, H), lambda i: (i, 0))]
    if with_xm:
        out_shape.append(jax.ShapeDtypeStruct((N, H // 2), jnp.uint32))
        out_specs.append(pl.BlockSpec((blk, H // 2), lambda i: (i, 0)))
    res = pl.pallas_call(
        functools.partial(_update_body, with_xm),
        grid=(grid,),
        in_specs=[
            pl.BlockSpec((blk, H), lambda i: (i, 0)),
            pl.BlockSpec((2, blk, H), lambda i: (0, i, 0)),
            pl.BlockSpec((H, H), lambda i: (0, 0)),
            pl.BlockSpec((H, H), lambda i: (0, 0)),
            pl.BlockSpec((1, H), lambda i: (0, 0)),
            pl.BlockSpec((1, H), lambda i: (0, 0)),
            pl.BlockSpec((1, H), lambda i: (0, 0)),
            pl.BlockSpec((H, H), lambda i: (0, 0)),
        ],
        out_specs=out_specs,
        out_shape=out_shape,
    )(x, agg2, Wut, Wub, bu, g, b, Wmt)
    return res if with_xm else (res[0], None)


def _pool_body(x, batch, Wp1, bp1, Wp2, bp2, out):
    gids = lax.broadcasted_iota(jnp.int32, (NUM_GRAPHS, 1), 0)
    A = (batch[...] == gids).astype(_F32)
    pooled = jnp.dot(A, x[...])
    counts = jnp.sum(A, axis=1, keepdims=True)
    pooled = pooled / jnp.maximum(counts, 1.0)
    h = jnp.maximum(jnp.dot(pooled, Wp1[...]) + bp1[...], 0.0)
    out[...] = jnp.dot(h, Wp2[...]) + bp2[...]


def _pool(x, batch2d, Wp1, bp1, Wp2, bp2):
    return pl.pallas_call(
        _pool_body,
        out_shape=jax.ShapeDtypeStruct((NUM_GRAPHS, H), _F32),
    )(x, batch2d, Wp1, bp1, Wp2, bp2)


NB = 5
LA = 2
ZR = 160


def _edge_pass_body(xm_hbm, eam_hbm, idx_hbm, out_hbm,
                    src_v, dst_v, *scr):
    bufs = scr[0:NB]
    gats = scr[NB:2 * NB]
    mbufs = scr[2 * NB:3 * NB]
    zbuf = scr[3 * NB]
    agg_sh = scr[3 * NB + 1]
    sems_e = scr[3 * NB + 2:3 * NB + 2 + NB]
    sems_g = scr[3 * NB + 2 + NB:3 * NB + 2 + 2 * NB]
    sems_s = scr[3 * NB + 2 + 2 * NB:3 * NB + 2 + 3 * NB]

    cc = lax.axis_index("c")
    ss = lax.axis_index("s")
    wid = cc * 16 + ss

    pltpu.sync_copy(idx_hbm.at[0, wid], src_v)
    pltpu.sync_copy(idx_hbm.at[1, wid], dst_v)

    ebase = wid * (EPT // 4)

    def start_loads(c, j):
        pltpu.async_copy(eam_hbm.at[pl.ds(ebase + c * (CH // 4), CH // 4)],
                         bufs[j], sems_e[j])
        pltpu.async_copy(xm_hbm.at[src_v.at[c]], gats[j], sems_g[j])

    start_loads(0, 0)
    start_loads(1, 1)

    def zrow(r, carry):
        for q in range(H // 16):
            zbuf[r, pl.ds(q * 16, 16)] = jnp.zeros((16,), _F32)
        return carry
    lax.fori_loop(0, ZR, zrow, 0)
    for q in range(RPT // ZR):
        pltpu.sync_copy(zbuf, agg_sh.at[pl.ds(ss * RPT + q * ZR, ZR)])
    plsc.subcore_barrier()

    def kbody(k, carry):
        for j in range(NB):
            c = NB * k + j
            pltpu.make_async_copy(
                eam_hbm.at[pl.ds(ebase + c * (CH // 4), CH // 4)], bufs[j],
                sems_e[j]).wait()
            pltpu.make_async_copy(xm_hbm.at[src_v.at[c]], gats[j], sems_g[j]).wait()

            buf, gat, mb = bufs[j], gats[j], mbufs[j]

            @plsc.parallel_loop(0, CH // 4, unroll=4)
            def _(r4):
                for k in range(4):
                    r = k * (CH // 4) + r4
                    for q in range(H // 32):
                        ve = buf[r4, pl.ds(k * 32 + q * 16, 16)]
                        ae, be_ = plsc.unpack(
                            plsc.bitcast(ve, jnp.bfloat16),
                            format=plsc.PackFormat.INTERLEAVED)
                        vg = gat[r, pl.ds(q * 16, 16)]
                        ag, bg = plsc.unpack(
                            plsc.bitcast(vg, jnp.bfloat16),
                            format=plsc.PackFormat.INTERLEAVED)
                        s1 = pl.ds(q * 32, 16)
                        s2 = pl.ds(q * 32 + 16, 16)
                        mb[r, s1] = jnp.maximum(ae + ag, 0.0)
                        mb[r, s2] = jnp.maximum(be_ + bg, 0.0)

            pltpu.async_copy(mb, agg_sh.at[dst_v.at[c]], sems_s[j], add=True)

            c2 = c + LA
            j2 = (j + LA) % NB

            @pl.when(c2 < NCH)
            def _():
                @pl.when(c2 >= NB)
                def _():
                    pltpu.make_async_copy(
                        bufs[j2], agg_sh.at[dst_v.at[c2]], sems_s[j2]).wait()
                start_loads(c2, j2)
        return carry
    lax.fori_loop(0, NCH // NB, kbody, 0)

    for j in range(NB):
        pltpu.make_async_copy(mbufs[j], agg_sh.at[dst_v.at[0]],
                              sems_s[j]).wait()

    plsc.subcore_barrier()
    for q in range(RPT // ZR):
        pltpu.sync_copy(agg_sh.at[pl.ds(ss * RPT + q * ZR, ZR)],
                        out_hbm.at[cc, pl.ds(ss * RPT + q * ZR, ZR)])


def _edge_pass(xm, eam_l, idx4):
    mesh = plsc.VectorSubcoreMesh(core_axis_name="c", subcore_axis_name="s")
    kern = pl.kernel(
        _edge_pass_body,
        out_type=jax.ShapeDtypeStruct((2, NP, H), _F32),
        mesh=mesh,
        scratch_types=(
            [pltpu.VMEM((NCH, CH), jnp.int32)] * 2
            + [pltpu.VMEM((CH // 4, 2 * H), jnp.uint32)] * NB
            + [pltpu.VMEM((CH, H // 2), jnp.uint32)] * NB
            + [pltpu.VMEM((CH, H), _F32)] * NB
            + [pltpu.VMEM((ZR, H), _F32)]
            + [pltpu.VMEM_SHARED((NP, H), _F32)]
            + [pltpu.SemaphoreType.DMA] * (3 * NB)
        ),
        compiler_params=pltpu.CompilerParams(use_tc_tiling_on_sc=False, needs_layout_passes=False),
    )
    return kern(xm, eam_l, idx4)


def _psi():
    return jnp.asarray(
        list(range(0, 16)) + list(range(32, 48))
        + list(range(16, 32)) + list(range(48, 64)), dtype=jnp.int32)


def kernel(node_feats, edge_feats, edge_index, batch,
           Wn, bn, We, be, Wm, bm, Wu, bu, ln_g, ln_b,
           Wp1, bp1, Wp2, bp2):
    psi = _psi()
    Wm_top = Wm[:, :H, :][:, :, psi]
    Wm_bot = Wm[:, H:, :][:, :, psi]
    bm_p = bm[:, psi]
    import jax.scipy.linalg as jsl
    Wfold = jnp.einsum("kh,lhj->lkj", We, Wm_bot)
    bfold = jnp.einsum("h,lhj->lj", be, Wm_bot) + bm_p
    Wf4, bf4 = [], []
    for l in range(L):
        lo = Wfold[l][:, :H // 2]
        hi = Wfold[l][:, H // 2:]
        Wf4.append(jnp.concatenate(
            [jsl.block_diag(lo, lo, lo, lo), jsl.block_diag(hi, hi, hi, hi)],
            axis=1))
        bf4.append(jnp.concatenate(
            [jnp.tile(bfold[l][:H // 2], 4), jnp.tile(bfold[l][H // 2:], 4)]))
    be2 = be.reshape(1, H)
    bn2 = bn.reshape(1, H)

    quarter = E // 4
    slabs = [edge_index[:, k * quarter:(k + 1) * quarter]
             .reshape(2, NW, NCH, CH // 4) for k in range(4)]
    idx4 = jnp.concatenate(slabs, axis=3)
    batch2d = batch.reshape(1, N)
    eft = edge_feats.T

    x, xm = _prologue(node_feats, Wn, bn2, Wm_top[0])
    eams = [_eam_layer(eft, Wf4[l], bf4[l].reshape(1, 4 * H))
            for l in range(L)]

    for l in range(L):
        agg2 = _edge_pass(xm, eams[l], idx4)
        with_xm = l < L - 1
        Wmt = Wm_top[l + 1] if with_xm else Wm_top[0]
        x, xm = _update(
            x, agg2,
            Wu[l, :H, :], Wu[l, H:, :], bu[l].reshape(1, H),
            ln_g[l].reshape(1, H), ln_b[l].reshape(1, H),
            Wmt, with_xm)

    return _pool(x, batch2d, Wp1.astype(_F32), bp1.reshape(1, H),
                 Wp2, bp2.reshape(1, H))


# speedup vs baseline: 2.0209x; 1.1068x over previous
import functools

import jax
import jax.numpy as jnp
from jax import lax
from jax.experimental import pallas as pl
from jax.experimental.pallas import tpu as pltpu
from jax.experimental.pallas import tpu_sc as plsc

N = 10000
E = 320000
ATOM_DIM = 128
BOND_DIM = 16
H = 64
L = 3
NUM_GRAPHS = 64

NW = 32
EPT = E // NW
CH = 80
NCH = EPT // CH
NP = 10240
RPT = NP // 16

_F32 = jnp.float32


def _prologue_body(nf, Wn, bn, Wm0t, x_out, xm_out):
    x = jnp.dot(nf[...], Wn[...]) + bn[...]
    x_out[...] = x
    xm_out[...] = _pack32(jnp.dot(x, Wm0t[...]))


def _prologue(node_feats, Wn, bn, Wm0t):
    return pl.pallas_call(
        _prologue_body,
        out_shape=(
            jax.ShapeDtypeStruct((N, H), _F32),
            jax.ShapeDtypeStruct((N, H // 2), jnp.uint32),
        ),
    )(node_feats, Wn, bn, Wm0t)


def _rn16(x):
    xi = lax.bitcast_convert_type(x, jnp.uint32)
    return xi + jnp.uint32(0x8000)


def _pack32(m):
    wlo = lax.shift_right_logical(_rn16(m[:, :H // 2]), jnp.uint32(16))
    return wlo | (_rn16(m[:, H // 2:]) & jnp.uint32(0xFFFF0000))


def _eam_body(ef0, ef1, ef2, ef3, Wf4, bf4, out):
    dn = (((0,), (0,)), ((), ()))
    lhs = jnp.concatenate([ef0[...], ef1[...], ef2[...], ef3[...]], axis=0)
    m4 = lax.dot_general(lhs, Wf4[...], dn,
                         preferred_element_type=_F32) + bf4[...]
    wlo = lax.shift_right_logical(_rn16(m4[:, :2 * H]), jnp.uint32(16))
    whi = _rn16(m4[:, 2 * H:]) & jnp.uint32(0xFFFF0000)
    out[...] = wlo | whi


def _eam_layer(eft, Wf_l, bf_l):
    blk = 3200
    grid = (E // 4) // blk
    return pl.pallas_call(
        _eam_body,
        grid=(grid,),
        in_specs=[
            pl.BlockSpec((BOND_DIM, blk), lambda i, k=k, g=grid: (0, i + k * g))
            for k in range(4)
        ] + [
            pl.BlockSpec((4 * BOND_DIM, 4 * H), lambda i: (0, 0)),
            pl.BlockSpec((1, 4 * H), lambda i: (0, 0)),
        ],
        out_specs=pl.BlockSpec((blk, 2 * H), lambda i: (i, 0)),
        out_shape=jax.ShapeDtypeStruct((E // 4, 2 * H), jnp.uint32),
    )(eft, eft, eft, eft, Wf_l, bf_l)


def _update_body(with_xm, x, agg2, Wut, Wub, bu, g, b, Wmt, xo, xmo=None):
    agg = agg2[0] + agg2[1]
    xv = x[...]
    xn = jnp.dot(xv, Wut[...]) + jnp.dot(agg, Wub[...]) + bu[...]
    y = xv + xn
    mu = jnp.mean(y, axis=-1, keepdims=True)
    d = y - mu
    var = jnp.mean(d * d, axis=-1, keepdims=True)
    xh = d * lax.rsqrt(var + 1e-5) * g[...] + b[...]
    xo[...] = xh
    if with_xm:
        xmo[...] = _pack32(jnp.dot(xh, Wmt[...]))


def _update(x, agg2, Wut, Wub, bu, g, b, Wmt, with_xm):
    blk = 2000
    grid = N // blk
    out_shape = [jax.ShapeDtypeStruct((N, H), _F32)]
    out_specs = [pl.BlockSpec((blk, H), lambda i: (i, 0))]
    if with_xm:
        out_shape.append(jax.ShapeDtypeStruct((N, H // 2), jnp.uint32))
        out_specs.append(pl.BlockSpec((blk, H // 2), lambda i: (i, 0)))
    res = pl.pallas_call(
        functools.partial(_update_body, with_xm),
        grid=(grid,),
        in_specs=[
            pl.BlockSpec((blk, H), lambda i: (i, 0)),
            pl.BlockSpec((2, blk, H), lambda i: (0, i, 0)),
            pl.BlockSpec((H, H), lambda i: (0, 0)),
            pl.BlockSpec((H, H), lambda i: (0, 0)),
            pl.BlockSpec((1, H), lambda i: (0, 0)),
            pl.BlockSpec((1, H), lambda i: (0, 0)),
            pl.BlockSpec((1, H), lambda i: (0, 0)),
            pl.BlockSpec((H, H), lambda i: (0, 0)),
        ],
        out_specs=out_specs,
        out_shape=out_shape,
    )(x, agg2, Wut, Wub, bu, g, b, Wmt)
    return res if with_xm else (res[0], None)


def _pool_body(x, batch, Wp1, bp1, Wp2, bp2, out):
    gids = lax.broadcasted_iota(jnp.int32, (NUM_GRAPHS, 1), 0)
    A = (batch[...] == gids).astype(_F32)
    pooled = jnp.dot(A, x[...])
    counts = jnp.sum(A, axis=1, keepdims=True)
    pooled = pooled / jnp.maximum(counts, 1.0)
    h = jnp.maximum(jnp.dot(pooled, Wp1[...]) + bp1[...], 0.0)
    out[...] = jnp.dot(h, Wp2[...]) + bp2[...]


def _pool(x, batch2d, Wp1, bp1, Wp2, bp2):
    return pl.pallas_call(
        _pool_body,
        out_shape=jax.ShapeDtypeStruct((NUM_GRAPHS, H), _F32),
    )(x, batch2d, Wp1, bp1, Wp2, bp2)


NB = 5
LA = 3
ZR = 160


def _edge_pass_body(xm_hbm, eam_hbm, idx_hbm, out_hbm,
                    src_v, dst_v, *scr):
    bufs = scr[0:NB]
    gats = scr[NB:2 * NB]
    mbufs = scr[2 * NB:3 * NB]
    zbuf = scr[3 * NB]
    agg_sh = scr[3 * NB + 1]
    sems_e = scr[3 * NB + 2:3 * NB + 2 + NB]
    sems_g = scr[3 * NB + 2 + NB:3 * NB + 2 + 2 * NB]
    sems_s = scr[3 * NB + 2 + 2 * NB:3 * NB + 2 + 3 * NB]

    cc = lax.axis_index("c")
    ss = lax.axis_index("s")
    wid = cc * 16 + ss

    pltpu.sync_copy(idx_hbm.at[0, wid], src_v)
    pltpu.sync_copy(idx_hbm.at[1, wid], dst_v)

    ebase = wid * (EPT // 4)

    def start_loads(c, j):
        pltpu.async_copy(eam_hbm.at[pl.ds(ebase + c * (CH // 4), CH // 4)],
                         bufs[j], sems_e[j])
        pltpu.async_copy(xm_hbm.at[src_v.at[c]], gats[j], sems_g[j])

    for p in range(LA):
        start_loads(p, p)

    def zrow(r, carry):
        for q in range(H // 16):
            zbuf[r, pl.ds(q * 16, 16)] = jnp.zeros((16,), _F32)
        return carry
    lax.fori_loop(0, ZR, zrow, 0)
    for q in range(RPT // ZR):
        pltpu.sync_copy(zbuf, agg_sh.at[pl.ds(ss * RPT + q * ZR, ZR)])
    plsc.subcore_barrier()

    def kbody(k, carry):
        for j in range(NB):
            c = NB * k + j
            pltpu.make_async_copy(
                eam_hbm.at[pl.ds(ebase + c * (CH // 4), CH // 4)], bufs[j],
                sems_e[j]).wait()
            pltpu.make_async_copy(xm_hbm.at[src_v.at[c]], gats[j], sems_g[j]).wait()

            buf, gat, mb = bufs[j], gats[j], mbufs[j]

            @plsc.parallel_loop(0, CH // 4, unroll=4)
            def _(r4):
                for k in range(4):
                    r = k * (CH // 4) + r4
                    for q in range(H // 32):
                        ve = buf[r4, pl.ds(k * 32 + q * 16, 16)]
                        ae, be_ = plsc.unpack(
                            plsc.bitcast(ve, jnp.bfloat16),
                            format=plsc.PackFormat.INTERLEAVED)
                        vg = gat[r, pl.ds(q * 16, 16)]
                        ag, bg = plsc.unpack(
                            plsc.bitcast(vg, jnp.bfloat16),
                            format=plsc.PackFormat.INTERLEAVED)
                        s1 = pl.ds(q * 32, 16)
                        s2 = pl.ds(q * 32 + 16, 16)
                        mb[r, s1] = jnp.maximum(ae + ag, 0.0)
                        mb[r, s2] = jnp.maximum(be_ + bg, 0.0)

            pltpu.async_copy(mb, agg_sh.at[dst_v.at[c]], sems_s[j], add=True)

            c2 = c + LA
            j2 = (j + LA) % NB

            @pl.when(c2 < NCH)
            def _():
                @pl.when(c2 >= NB)
                def _():
                    pltpu.make_async_copy(
                        bufs[j2], agg_sh.at[dst_v.at[c2]], sems_s[j2]).wait()
                start_loads(c2, j2)
        return carry
    lax.fori_loop(0, NCH // NB, kbody, 0)

    for j in range(NB):
        pltpu.make_async_copy(mbufs[j], agg_sh.at[dst_v.at[0]],
                              sems_s[j]).wait()

    plsc.subcore_barrier()
    for q in range(RPT // ZR):
        pltpu.sync_copy(agg_sh.at[pl.ds(ss * RPT + q * ZR, ZR)],
                        out_hbm.at[cc, pl.ds(ss * RPT + q * ZR, ZR)])


def _edge_pass(xm, eam_l, idx4):
    mesh = plsc.VectorSubcoreMesh(core_axis_name="c", subcore_axis_name="s")
    kern = pl.kernel(
        _edge_pass_body,
        out_type=jax.ShapeDtypeStruct((2, NP, H), _F32),
        mesh=mesh,
        scratch_types=(
            [pltpu.VMEM((NCH, CH), jnp.int32)] * 2
            + [pltpu.VMEM((CH // 4, 2 * H), jnp.uint32)] * NB
            + [pltpu.VMEM((CH, H // 2), jnp.uint32)] * NB
            + [pltpu.VMEM((CH, H), _F32)] * NB
            + [pltpu.VMEM((ZR, H), _F32)]
            + [pltpu.VMEM_SHARED((NP, H), _F32)]
            + [pltpu.SemaphoreType.DMA] * (3 * NB)
        ),
        compiler_params=pltpu.CompilerParams(use_tc_tiling_on_sc=False, needs_layout_passes=False),
    )
    return kern(xm, eam_l, idx4)


def _psi():
    return jnp.asarray(
        list(range(0, 16)) + list(range(32, 48))
        + list(range(16, 32)) + list(range(48, 64)), dtype=jnp.int32)


def kernel(node_feats, edge_feats, edge_index, batch,
           Wn, bn, We, be, Wm, bm, Wu, bu, ln_g, ln_b,
           Wp1, bp1, Wp2, bp2):
    psi = _psi()
    Wm_top = Wm[:, :H, :][:, :, psi]
    Wm_bot = Wm[:, H:, :][:, :, psi]
    bm_p = bm[:, psi]
    import jax.scipy.linalg as jsl
    Wfold = jnp.einsum("kh,lhj->lkj", We, Wm_bot)
    bfold = jnp.einsum("h,lhj->lj", be, Wm_bot) + bm_p
    Wf4, bf4 = [], []
    for l in range(L):
        lo = Wfold[l][:, :H // 2]
        hi = Wfold[l][:, H // 2:]
        Wf4.append(jnp.concatenate(
            [jsl.block_diag(lo, lo, lo, lo), jsl.block_diag(hi, hi, hi, hi)],
            axis=1))
        bf4.append(jnp.concatenate(
            [jnp.tile(bfold[l][:H // 2], 4), jnp.tile(bfold[l][H // 2:], 4)]))
    be2 = be.reshape(1, H)
    bn2 = bn.reshape(1, H)

    quarter = E // 4
    slabs = [edge_index[:, k * quarter:(k + 1) * quarter]
             .reshape(2, NW, NCH, CH // 4) for k in range(4)]
    idx4 = jnp.concatenate(slabs, axis=3)
    batch2d = batch.reshape(1, N)
    eft = edge_feats.T

    x, xm = _prologue(node_feats, Wn, bn2, Wm_top[0])
    eams = [_eam_layer(eft, Wf4[l], bf4[l].reshape(1, 4 * H))
            for l in range(L)]

    for l in range(L):
        agg2 = _edge_pass(xm, eams[l], idx4)
        with_xm = l < L - 1
        Wmt = Wm_top[l + 1] if with_xm else Wm_top[0]
        x, xm = _update(
            x, agg2,
            Wu[l, :H, :], Wu[l, H:, :], bu[l].reshape(1, H),
            ln_g[l].reshape(1, H), ln_b[l].reshape(1, H),
            Wmt, with_xm)

    return _pool(x, batch2d, Wp1.astype(_F32), bp1.reshape(1, H),
                 Wp2, bp2.reshape(1, H))


# speedup vs baseline: 2.1003x; 1.0393x over previous
import functools

import jax
import jax.numpy as jnp
from jax import lax
from jax.experimental import pallas as pl
from jax.experimental.pallas import tpu as pltpu
from jax.experimental.pallas import tpu_sc as plsc

N = 10000
E = 320000
ATOM_DIM = 128
BOND_DIM = 16
H = 64
L = 3
NUM_GRAPHS = 64

NW = 32
EPT = E // NW
CH = 80
NCH = EPT // CH
NP = 10240
RPT = NP // 16

_F32 = jnp.float32


def _prologue_body(nf, Wn, bn, Wm0t, x_out, xm_out):
    x = jnp.dot(nf[...], Wn[...]) + bn[...]
    x_out[...] = x
    xm_out[...] = _pack32(jnp.dot(x, Wm0t[...]))


def _prologue(node_feats, Wn, bn, Wm0t):
    return pl.pallas_call(
        _prologue_body,
        out_shape=(
            jax.ShapeDtypeStruct((N, H), _F32),
            jax.ShapeDtypeStruct((N, H // 2), jnp.uint32),
        ),
    )(node_feats, Wn, bn, Wm0t)


def _rn16(x):
    xi = lax.bitcast_convert_type(x, jnp.uint32)
    return xi + jnp.uint32(0x8000)


def _pack32(m):
    wlo = lax.shift_right_logical(_rn16(m[:, :H // 2]), jnp.uint32(16))
    return wlo | (_rn16(m[:, H // 2:]) & jnp.uint32(0xFFFF0000))


def _eam_body(ef0, ef1, ef2, ef3, Wf4, bf4, out):
    dn = (((0,), (0,)), ((), ()))
    lhs = jnp.concatenate([ef0[...], ef1[...], ef2[...], ef3[...]], axis=0)
    m4 = lax.dot_general(lhs, Wf4[...], dn,
                         preferred_element_type=_F32) + bf4[...]
    wlo = lax.shift_right_logical(_rn16(m4[:, :2 * H]), jnp.uint32(16))
    whi = _rn16(m4[:, 2 * H:]) & jnp.uint32(0xFFFF0000)
    out[...] = wlo | whi


def _eam_layer(eft, Wf_l, bf_l):
    blk = 3200
    grid = (E // 4) // blk
    return pl.pallas_call(
        _eam_body,
        grid=(grid,),
        in_specs=[
            pl.BlockSpec((BOND_DIM, blk), lambda i, k=k, g=grid: (0, i + k * g))
            for k in range(4)
        ] + [
            pl.BlockSpec((4 * BOND_DIM, 4 * H), lambda i: (0, 0)),
            pl.BlockSpec((1, 4 * H), lambda i: (0, 0)),
        ],
        out_specs=pl.BlockSpec((blk, 2 * H), lambda i: (i, 0)),
        out_shape=jax.ShapeDtypeStruct((E // 4, 2 * H), jnp.uint32),
    )(eft, eft, eft, eft, Wf_l, bf_l)


def _update_body(with_xm, x, agg2, Wut, Wub, bu, g, b, Wmt, xo, xmo=None):
    agg = agg2[0] + agg2[1]
    xv = x[...]
    xn = jnp.dot(xv, Wut[...]) + jnp.dot(agg, Wub[...]) + bu[...]
    y = xv + xn
    mu = jnp.mean(y, axis=-1, keepdims=True)
    d = y - mu
    var = jnp.mean(d * d, axis=-1, keepdims=True)
    xh = d * lax.rsqrt(var + 1e-5) * g[...] + b[...]
    xo[...] = xh
    if with_xm:
        xmo[...] = _pack32(jnp.dot(xh, Wmt[...]))


def _update(x, agg2, Wut, Wub, bu, g, b, Wmt, with_xm):
    blk = 2000
    grid = N // blk
    out_shape = [jax.ShapeDtypeStruct((N, H), _F32)]
    out_specs = [pl.BlockSpec((blk, H), lambda i: (i, 0))]
    if with_xm:
        out_shape.append(jax.ShapeDtypeStruct((N, H // 2), jnp.uint32))
        out_specs.append(pl.BlockSpec((blk, H // 2), lambda i: (i, 0)))
    res = pl.pallas_call(
        functools.partial(_update_body, with_xm),
        grid=(grid,),
        in_specs=[
            pl.BlockSpec((blk, H), lambda i: (i, 0)),
            pl.BlockSpec((2, blk, H), lambda i: (0, i, 0)),
            pl.BlockSpec((H, H), lambda i: (0, 0)),
            pl.BlockSpec((H, H), lambda i: (0, 0)),
            pl.BlockSpec((1, H), lambda i: (0, 0)),
            pl.BlockSpec((1, H), lambda i: (0, 0)),
            pl.BlockSpec((1, H), lambda i: (0, 0)),
            pl.BlockSpec((H, H), lambda i: (0, 0)),
        ],
        out_specs=out_specs,
        out_shape=out_shape,
    )(x, agg2, Wut, Wub, bu, g, b, Wmt)
    return res if with_xm else (res[0], None)


def _pool_body(x, batch, Wp1, bp1, Wp2, bp2, out):
    gids = lax.broadcasted_iota(jnp.int32, (NUM_GRAPHS, 1), 0)
    A = (batch[...] == gids).astype(_F32)
    pooled = jnp.dot(A, x[...])
    counts = jnp.sum(A, axis=1, keepdims=True)
    pooled = pooled / jnp.maximum(counts, 1.0)
    h = jnp.maximum(jnp.dot(pooled, Wp1[...]) + bp1[...], 0.0)
    out[...] = jnp.dot(h, Wp2[...]) + bp2[...]


def _pool(x, batch2d, Wp1, bp1, Wp2, bp2):
    return pl.pallas_call(
        _pool_body,
        out_shape=jax.ShapeDtypeStruct((NUM_GRAPHS, H), _F32),
    )(x, batch2d, Wp1, bp1, Wp2, bp2)


NB = 5
LA = 4
ZR = 160


def _edge_pass_body(xm_hbm, eam_hbm, idx_hbm, out_hbm,
                    src_v, dst_v, *scr):
    bufs = scr[0:NB]
    gats = scr[NB:2 * NB]
    mbufs = scr[2 * NB:3 * NB]
    zbuf = scr[3 * NB]
    agg_sh = scr[3 * NB + 1]
    sems_e = scr[3 * NB + 2:3 * NB + 2 + NB]
    sems_g = scr[3 * NB + 2 + NB:3 * NB + 2 + 2 * NB]
    sems_s = scr[3 * NB + 2 + 2 * NB:3 * NB + 2 + 3 * NB]

    cc = lax.axis_index("c")
    ss = lax.axis_index("s")
    wid = cc * 16 + ss

    pltpu.sync_copy(idx_hbm.at[0, wid], src_v)
    pltpu.sync_copy(idx_hbm.at[1, wid], dst_v)

    ebase = wid * (EPT // 4)

    def start_loads(c, j):
        pltpu.async_copy(eam_hbm.at[pl.ds(ebase + c * (CH // 4), CH // 4)],
                         bufs[j], sems_e[j])
        pltpu.async_copy(xm_hbm.at[src_v.at[c]], gats[j], sems_g[j])

    for p in range(LA):
        start_loads(p, p)

    def zrow(r, carry):
        for q in range(H // 16):
            zbuf[r, pl.ds(q * 16, 16)] = jnp.zeros((16,), _F32)
        return carry
    lax.fori_loop(0, ZR, zrow, 0)
    for q in range(RPT // ZR):
        pltpu.sync_copy(zbuf, agg_sh.at[pl.ds(ss * RPT + q * ZR, ZR)])
    plsc.subcore_barrier()

    def kbody(k, carry):
        for j in range(NB):
            c = NB * k + j
            pltpu.make_async_copy(
                eam_hbm.at[pl.ds(ebase + c * (CH // 4), CH // 4)], bufs[j],
                sems_e[j]).wait()
            pltpu.make_async_copy(xm_hbm.at[src_v.at[c]], gats[j], sems_g[j]).wait()

            buf, gat, mb = bufs[j], gats[j], mbufs[j]

            @plsc.parallel_loop(0, CH // 4, unroll=4)
            def _(r4):
                for k in range(4):
                    r = k * (CH // 4) + r4
                    for q in range(H // 32):
                        ve = buf[r4, pl.ds(k * 32 + q * 16, 16)]
                        ae, be_ = plsc.unpack(
                            plsc.bitcast(ve, jnp.bfloat16),
                            format=plsc.PackFormat.INTERLEAVED)
                        vg = gat[r, pl.ds(q * 16, 16)]
                        ag, bg = plsc.unpack(
                            plsc.bitcast(vg, jnp.bfloat16),
                            format=plsc.PackFormat.INTERLEAVED)
                        s1 = pl.ds(q * 32, 16)
                        s2 = pl.ds(q * 32 + 16, 16)
                        mb[r, s1] = jnp.maximum(ae + ag, 0.0)
                        mb[r, s2] = jnp.maximum(be_ + bg, 0.0)

            pltpu.async_copy(mb, agg_sh.at[dst_v.at[c]], sems_s[j], add=True)

            c2 = c + LA
            j2 = (j + LA) % NB

            @pl.when(c2 < NCH)
            def _():
                @pl.when(c2 >= NB)
                def _():
                    pltpu.make_async_copy(
                        bufs[j2], agg_sh.at[dst_v.at[c2]], sems_s[j2]).wait()
                start_loads(c2, j2)
        return carry
    lax.fori_loop(0, NCH // NB, kbody, 0)

    for j in range(NB):
        pltpu.make_async_copy(mbufs[j], agg_sh.at[dst_v.at[0]],
                              sems_s[j]).wait()

    plsc.subcore_barrier()
    for q in range(RPT // ZR):
        pltpu.sync_copy(agg_sh.at[pl.ds(ss * RPT + q * ZR, ZR)],
                        out_hbm.at[cc, pl.ds(ss * RPT + q * ZR, ZR)])


def _edge_pass(xm, eam_l, idx4):
    mesh = plsc.VectorSubcoreMesh(core_axis_name="c", subcore_axis_name="s")
    kern = pl.kernel(
        _edge_pass_body,
        out_type=jax.ShapeDtypeStruct((2, NP, H), _F32),
        mesh=mesh,
        scratch_types=(
            [pltpu.VMEM((NCH, CH), jnp.int32)] * 2
            + [pltpu.VMEM((CH // 4, 2 * H), jnp.uint32)] * NB
            + [pltpu.VMEM((CH, H // 2), jnp.uint32)] * NB
            + [pltpu.VMEM((CH, H), _F32)] * NB
            + [pltpu.VMEM((ZR, H), _F32)]
            + [pltpu.VMEM_SHARED((NP, H), _F32)]
            + [pltpu.SemaphoreType.DMA] * (3 * NB)
        ),
        compiler_params=pltpu.CompilerParams(use_tc_tiling_on_sc=False, needs_layout_passes=False),
    )
    return kern(xm, eam_l, idx4)


def _psi():
    return jnp.asarray(
        list(range(0, 16)) + list(range(32, 48))
        + list(range(16, 32)) + list(range(48, 64)), dtype=jnp.int32)


def kernel(node_feats, edge_feats, edge_index, batch,
           Wn, bn, We, be, Wm, bm, Wu, bu, ln_g, ln_b,
           Wp1, bp1, Wp2, bp2):
    psi = _psi()
    Wm_top = Wm[:, :H, :][:, :, psi]
    Wm_bot = Wm[:, H:, :][:, :, psi]
    bm_p = bm[:, psi]
    import jax.scipy.linalg as jsl
    Wfold = jnp.einsum("kh,lhj->lkj", We, Wm_bot)
    bfold = jnp.einsum("h,lhj->lj", be, Wm_bot) + bm_p
    Wf4, bf4 = [], []
    for l in range(L):
        lo = Wfold[l][:, :H // 2]
        hi = Wfold[l][:, H // 2:]
        Wf4.append(jnp.concatenate(
            [jsl.block_diag(lo, lo, lo, lo), jsl.block_diag(hi, hi, hi, hi)],
            axis=1))
        bf4.append(jnp.concatenate(
            [jnp.tile(bfold[l][:H // 2], 4), jnp.tile(bfold[l][H // 2:], 4)]))
    be2 = be.reshape(1, H)
    bn2 = bn.reshape(1, H)

    quarter = E // 4
    slabs = [edge_index[:, k * quarter:(k + 1) * quarter]
             .reshape(2, NW, NCH, CH // 4) for k in range(4)]
    idx4 = jnp.concatenate(slabs, axis=3)
    batch2d = batch.reshape(1, N)
    eft = edge_feats.T

    x, xm = _prologue(node_feats, Wn, bn2, Wm_top[0])
    eams = [_eam_layer(eft, Wf4[l], bf4[l].reshape(1, 4 * H))
            for l in range(L)]

    for l in range(L):
        agg2 = _edge_pass(xm, eams[l], idx4)
        with_xm = l < L - 1
        Wmt = Wm_top[l + 1] if with_xm else Wm_top[0]
        x, xm = _update(
            x, agg2,
            Wu[l, :H, :], Wu[l, H:, :], bu[l].reshape(1, H),
            ln_g[l].reshape(1, H), ln_b[l].reshape(1, H),
            Wmt, with_xm)

    return _pool(x, batch2d, Wp1.astype(_F32), bp1.reshape(1, H),
                 Wp2, bp2.reshape(1, H))
